# baseline (speedup 1.0000x reference)
import dataclasses

import jax
import jax.numpy as jnp
from jax import lax
from jax.experimental import pallas as pl
from jax.experimental.pallas import tpu as pltpu
from jax.experimental.pallas import tpu_sc as plsc

NN = 10000
NP = 10240
RPT = 640
EE = 320000
NT = 16
NB = 160
BE = 128
EP = NT * NB * BE
SB = 16
NSB = NB // SB
DIN = 128
HH = 512
CC = 64
CW = 128
NCK = HH // CW


def _agg_body(cfg_h, zcat_h, src_h, dst_h, zer_h, out_h, cfg_s, si0, si1,
              didx_v, b0, b1, acc_s, s0, s1, is0, is1, ss0, ss1):
  bufs = (b0, b1)
  gsems = (s0, s1)
  ssems = (ss0, ss1)
  sidx = (si0, si1)
  isems = (is0, is1)
  core = lax.axis_index("c")
  tid = lax.axis_index("s")
  row0 = tid * RPT

  def stream_superblock(slot, sb, par, hi):
    sv = sidx[par]
    pltpu.async_copy(zcat_h.at[sv.at[0]], bufs[0], gsems[0])

    @pl.when(sb + 1 < hi)
    def _():
      pltpu.async_copy(src_h.at[slot, tid, pl.ds((sb + 1) * SB, SB)],
                       sidx[1 - par], isems[1 - par])

    pltpu.sync_copy(dst_h.at[slot, tid, pl.ds(sb * SB, SB)], didx_v)
    for k in range(SB):
      b = k % 2
      pltpu.make_async_copy(zcat_h.at[sv.at[0]], bufs[b], gsems[b]).wait()
      pltpu.async_copy(bufs[b], acc_s.at[didx_v.at[k]], ssems[b], add=True)
      if k + 1 < SB:
        if k >= 1:
          pltpu.make_async_copy(bufs[1 - b], acc_s.at[didx_v.at[0]],
                                ssems[1 - b]).wait()
        pltpu.async_copy(zcat_h.at[sv.at[k + 1]], bufs[1 - b], gsems[1 - b])
    for b in range(2):
      pltpu.make_async_copy(bufs[b], acc_s.at[didx_v.at[0]], ssems[b]).wait()

  pltpu.sync_copy(cfg_h, cfg_s)
  for ci in range(2):
    slot = core * 2 + ci
    lo = jnp.max(cfg_s[slot, 0])
    hi = jnp.max(cfg_s[slot, 1])
    pltpu.sync_copy(zer_h, acc_s.at[pl.ds(row0, RPT)])
    plsc.subcore_barrier()

    @pl.when(hi > lo)
    def _():
      pltpu.sync_copy(src_h.at[slot, tid, pl.ds(lo * SB, SB)], sidx[0])

    @pl.loop(0, (hi - lo) // 2)
    def _(p):
      for par in range(2):
        sb = lo + 2 * p + par

        @pl.when(sb > lo)
        def _():
          pltpu.make_async_copy(src_h.at[slot, tid, pl.ds(0, SB)],
                                sidx[par], isems[par]).wait()

        stream_superblock(slot, sb, par, hi)

    @pl.when(jnp.logical_and(hi > lo, (hi - lo) % 2 == 1))
    def _():
      sb = hi - 1

      @pl.when(sb > lo)
      def _():
        pltpu.make_async_copy(src_h.at[slot, tid, pl.ds(0, SB)], sidx[0],
                              isems[0]).wait()

      stream_superblock(slot, sb, 0, hi)

    plsc.subcore_barrier()
    pltpu.sync_copy(acc_s.at[pl.ds(row0, RPT)],
                    out_h.at[slot, pl.ds(row0, RPT)])
    plsc.subcore_barrier()


def _sc_params():
  cp = pltpu.CompilerParams()
  if "needs_layout_passes" in pltpu.CompilerParams.__dataclass_fields__:
    cp = dataclasses.replace(cp, needs_layout_passes=False)
  return cp


def _edge_agg(cfg, zcat, src_t, dst_t, zer):
  kern = pl.kernel(
      _agg_body,
      compiler_params=_sc_params(),
      out_type=jax.ShapeDtypeStruct((NCK, NP, CW), jnp.float32),
      mesh=plsc.VectorSubcoreMesh(core_axis_name="c", subcore_axis_name="s"),
      scratch_types=[
          pltpu.VMEM((NCK, 2, 16), jnp.int32),
          pltpu.VMEM((SB, BE), jnp.int32),
          pltpu.VMEM((SB, BE), jnp.int32),
          pltpu.VMEM((SB, BE), jnp.int32),
          pltpu.VMEM((BE, CW), jnp.float32),
          pltpu.VMEM((BE, CW), jnp.float32),
          pltpu.VMEM_SHARED((NP, CW), jnp.float32),
          pltpu.SemaphoreType.DMA,
          pltpu.SemaphoreType.DMA,
          pltpu.SemaphoreType.DMA,
          pltpu.SemaphoreType.DMA,
          pltpu.SemaphoreType.DMA,
          pltpu.SemaphoreType.DMA,
      ],
  )
  return kern(cfg, zcat, src_t, dst_t, zer)


def _l1_body(x_ref, d_ref, w_ref, o_ref):
  ns = lax.rsqrt(jnp.maximum(d_ref[...], 1.0))
  z = jnp.dot(x_ref[...] * ns, w_ref[...], preferred_element_type=jnp.float32)
  for c in range(NCK):
    o_ref[c] = z[:, c * CW:(c + 1) * CW]


def _layer1(x_pad, dout_col, w1):
  return pl.pallas_call(
      _l1_body,
      grid=(NP // RPT,),
      in_specs=[
          pl.BlockSpec((RPT, DIN), lambda i: (i, 0)),
          pl.BlockSpec((RPT, 1), lambda i: (i, 0)),
          pl.BlockSpec((DIN, HH), lambda i: (0, 0)),
      ],
      out_specs=pl.BlockSpec((NCK, RPT, CW), lambda i: (0, i, 0)),
      out_shape=jax.ShapeDtypeStruct((NCK, NP, CW), jnp.float32),
  )(x_pad, dout_col, w1)


def _mid_body(m_ref, din_ref, dout_ref, b_ref, w_ref, o_ref):
  nd = lax.rsqrt(jnp.maximum(din_ref[...], 1.0))
  ns = lax.rsqrt(jnp.maximum(dout_ref[...], 1.0))
  m = jnp.concatenate([m_ref[c] for c in range(NCK)], axis=1)
  h = jnp.maximum(m * nd + b_ref[...], 0.0) * ns
  z = jnp.dot(h, w_ref[...], preferred_element_type=jnp.float32)
  for c in range(NCK):
    o_ref[c] = z[:, c * CW:(c + 1) * CW]


def _layer_mid(m_cat, din_col, dout_col, b_row, w):
  return pl.pallas_call(
      _mid_body,
      grid=(NP // RPT,),
      in_specs=[
          pl.BlockSpec((NCK, RPT, CW), lambda i: (0, i, 0)),
          pl.BlockSpec((RPT, 1), lambda i: (i, 0)),
          pl.BlockSpec((RPT, 1), lambda i: (i, 0)),
          pl.BlockSpec((1, HH), lambda i: (0, 0)),
          pl.BlockSpec((HH, HH), lambda i: (0, 0)),
      ],
      out_specs=pl.BlockSpec((NCK, RPT, CW), lambda i: (0, i, 0)),
      out_shape=jax.ShapeDtypeStruct((NCK, NP, CW), jnp.float32),
  )(m_cat, din_col, dout_col, b_row, w)


def _l4_body(m_ref, din_ref, dout_ref, b_ref, w_ref, o_ref):
  nd = lax.rsqrt(jnp.maximum(din_ref[...], 1.0))
  ns = lax.rsqrt(jnp.maximum(dout_ref[...], 1.0))
  m = jnp.concatenate([m_ref[c] for c in range(NCK)], axis=1)
  h = jnp.maximum(m * nd + b_ref[...], 0.0) * ns
  o_ref[...] = jnp.dot(h, w_ref[...], preferred_element_type=jnp.float32)


def _layer4(m_cat, din_col, dout_col, b_row, w4p):
  return pl.pallas_call(
      _l4_body,
      grid=(NP // RPT,),
      in_specs=[
          pl.BlockSpec((NCK, RPT, CW), lambda i: (0, i, 0)),
          pl.BlockSpec((RPT, 1), lambda i: (i, 0)),
          pl.BlockSpec((RPT, 1), lambda i: (i, 0)),
          pl.BlockSpec((1, HH), lambda i: (0, 0)),
          pl.BlockSpec((HH, CW), lambda i: (0, 0)),
      ],
      out_specs=pl.BlockSpec((RPT, CW), lambda i: (i, 0)),
      out_shape=jax.ShapeDtypeStruct((NP, CW), jnp.float32),
  )(m_cat, din_col, dout_col, b_row, w4p)


def _fin_body(p_ref, din_ref, b_ref, o_ref):
  nd = lax.rsqrt(jnp.maximum(din_ref[...], 1.0))
  p = p_ref[0] + p_ref[1] + p_ref[2] + p_ref[3]
  o_ref[...] = p[:, :CC] * nd + b_ref[...]


def _final(parts, din_col, b4_row):
  return pl.pallas_call(
      _fin_body,
      grid=(NP // RPT,),
      in_specs=[
          pl.BlockSpec((NCK, RPT, CW), lambda i: (0, i, 0)),
          pl.BlockSpec((RPT, 1), lambda i: (i, 0)),
          pl.BlockSpec((1, CC), lambda i: (0, 0)),
      ],
      out_specs=pl.BlockSpec((RPT, CC), lambda i: (i, 0)),
      out_shape=jax.ShapeDtypeStruct((NP, CC), jnp.float32),
  )(parts, din_col, b4_row)


@jax.jit
def kernel(x, edge_index, W1, b1, W2, b2, W3, b3, W4, b4):
  src = edge_index[0]
  dst = edge_index[1]
  pad = EP - EE

  src_deg = jnp.concatenate(
      [src, jnp.full((pad,), NN, jnp.int32)]).reshape(NT, NB, BE)
  dst_pad = jnp.concatenate(
      [dst, jnp.full((pad,), NN, jnp.int32)]).reshape(NT, NB, BE)
  src_pad = jnp.concatenate([src, jnp.zeros((pad,), jnp.int32)])
  offs = (jnp.arange(NCK, dtype=jnp.int32) * NP)[:, None]
  src4 = (src_pad[None, :] + offs).reshape(NCK, NT, NB, BE)
  src_fin = jnp.broadcast_to(src_pad.reshape(NT, NB, BE), (NCK, NT, NB, BE))
  dst4 = jnp.broadcast_to(dst_pad, (NCK, NT, NB, BE))

  zer = jnp.zeros((RPT, CW), jnp.float32)
  x_pad = jnp.concatenate([x, jnp.zeros((NP - NN, DIN), jnp.float32)])
  w4p = jnp.pad(W4, ((0, 0), (0, CW - CC)))

  def _cfg(rows):
    return jnp.broadcast_to(
        jnp.array(rows, jnp.int32)[:, :, None], (NCK, 2, 16))

  cfg_full = _cfg([[0, NSB]] * NCK)
  cfg_deg = _cfg([[0, NSB], [0, 0], [0, NSB], [0, 0]])
  cfg_fin = _cfg([[0, NSB // 2], [0, 0], [NSB // 2, NSB], [0, 0]])

  ones_cat = jnp.ones((NCK * NP, CW), jnp.float32)
  dst_deg4 = jnp.stack([src_deg, src_deg, dst_pad, dst_pad])
  deg = _edge_agg(cfg_deg, ones_cat, src4, dst_deg4, zer)
  dout_col = deg[0, :, 0:1]
  din_col = deg[2, :, 0:1]

  z1 = _layer1(x_pad, dout_col, W1).reshape(NCK * NP, CW)
  m1 = _edge_agg(cfg_full, z1, src4, dst4, zer)
  z2 = _layer_mid(m1, din_col, dout_col, b1.reshape(1, HH),
                  W2).reshape(NCK * NP, CW)
  m2 = _edge_agg(cfg_full, z2, src4, dst4, zer)
  z3 = _layer_mid(m2, din_col, dout_col, b2.reshape(1, HH),
                  W3).reshape(NCK * NP, CW)
  m3 = _edge_agg(cfg_full, z3, src4, dst4, zer)
  z4 = _layer4(m3, din_col, dout_col, b3.reshape(1, HH), w4p)
  z4cat = jnp.concatenate(
      [z4, jnp.zeros(((NCK - 1) * NP, CW), jnp.float32)])
  parts = _edge_agg(cfg_fin, z4cat, src_fin, dst4, zer)
  out = _final(parts, din_col, b4.reshape(1, CC))
  return out[:NN]


# speedup vs baseline: 1.0604x; 1.0604x over previous
import dataclasses

import jax
import jax.numpy as jnp
from jax import lax
from jax.experimental import pallas as pl
from jax.experimental.pallas import tpu as pltpu
from jax.experimental.pallas import tpu_sc as plsc

NN = 10000
NP = 10240
RPT = 640
EE = 320000
NT = 16
NB = 160
BE = 128
EP = NT * NB * BE
SB = 16
NSB = NB // SB
DIN = 128
HH = 512
CC = 64
CW = 128
NCK = HH // CW


def _agg_body(cfg_h, zcat_h, src_h, dst_h, zer_h, out_h, cfg_s, si0, si1,
              didx_v, b0, b1, acc_s, s0, s1, is0, is1):
  bufs = (b0, b1)
  sems = (s0, s1)
  sidx = (si0, si1)
  isems = (is0, is1)
  core = lax.axis_index("c")
  tid = lax.axis_index("s")
  row0 = tid * RPT

  pltpu.sync_copy(cfg_h, cfg_s)
  for ci in range(2):
    slot = core * 2 + ci
    lo = jnp.max(cfg_s[slot, 0])
    hi = jnp.max(cfg_s[slot, 1])
    pltpu.sync_copy(zer_h, acc_s.at[pl.ds(row0, RPT)])
    plsc.subcore_barrier()

    @pl.when(hi > lo)
    def _():
      pltpu.sync_copy(src_h.at[slot, tid, pl.ds(lo * SB, SB)], sidx[0])

    @pl.loop(0, (hi - lo) // 2)
    def _(p):
      for par in range(2):
        sb = lo + 2 * p + par

        @pl.when(sb > lo)
        def _():
          pltpu.make_async_copy(src_h.at[slot, tid, pl.ds(0, SB)],
                                sidx[par], isems[par]).wait()

        sv = sidx[par]
        for b in range(2):
          pltpu.async_copy(zcat_h.at[sv.at[b]], bufs[b], sems[b])

        @pl.when(sb + 1 < hi)
        def _():
          pltpu.async_copy(src_h.at[slot, tid, pl.ds((sb + 1) * SB, SB)],
                           sidx[1 - par], isems[1 - par])

        pltpu.sync_copy(dst_h.at[slot, tid, pl.ds(sb * SB, SB)], didx_v)
        for k in range(SB):
          b = k % 2
          pltpu.make_async_copy(zcat_h.at[sv.at[0]], bufs[b], sems[b]).wait()
          pltpu.sync_copy(bufs[b], acc_s.at[didx_v.at[k]], add=True)
          if k + 2 < SB:
            pltpu.async_copy(zcat_h.at[sv.at[k + 2]], bufs[b], sems[b])

    @pl.when(jnp.logical_and(hi > lo, (hi - lo) % 2 == 1))
    def _():
      sb = hi - 1

      @pl.when(sb > lo)
      def _():
        pltpu.make_async_copy(src_h.at[slot, tid, pl.ds(0, SB)], sidx[0],
                              isems[0]).wait()

      sv = sidx[0]
      for b in range(2):
        pltpu.async_copy(zcat_h.at[sv.at[b]], bufs[b], sems[b])
      pltpu.sync_copy(dst_h.at[slot, tid, pl.ds(sb * SB, SB)], didx_v)
      for k in range(SB):
        b = k % 2
        pltpu.make_async_copy(zcat_h.at[sv.at[0]], bufs[b], sems[b]).wait()
        pltpu.sync_copy(bufs[b], acc_s.at[didx_v.at[k]], add=True)
        if k + 2 < SB:
          pltpu.async_copy(zcat_h.at[sv.at[k + 2]], bufs[b], sems[b])

    plsc.subcore_barrier()
    pltpu.sync_copy(acc_s.at[pl.ds(row0, RPT)],
                    out_h.at[slot, pl.ds(row0, RPT)])
    plsc.subcore_barrier()


def _sc_params():
  cp = pltpu.CompilerParams()
  if "needs_layout_passes" in pltpu.CompilerParams.__dataclass_fields__:
    cp = dataclasses.replace(cp, needs_layout_passes=False)
  return cp


def _edge_agg(cfg, zcat, src_t, dst_t, zer):
  kern = pl.kernel(
      _agg_body,
      compiler_params=_sc_params(),
      out_type=jax.ShapeDtypeStruct((NCK, NP, CW), jnp.float32),
      mesh=plsc.VectorSubcoreMesh(core_axis_name="c", subcore_axis_name="s"),
      scratch_types=[
          pltpu.VMEM((NCK, 2, 16), jnp.int32),
          pltpu.VMEM((SB, BE), jnp.int32),
          pltpu.VMEM((SB, BE), jnp.int32),
          pltpu.VMEM((SB, BE), jnp.int32),
          pltpu.VMEM((BE, CW), jnp.float32),
          pltpu.VMEM((BE, CW), jnp.float32),
          pltpu.VMEM_SHARED((NP, CW), jnp.float32),
          pltpu.SemaphoreType.DMA,
          pltpu.SemaphoreType.DMA,
          pltpu.SemaphoreType.DMA,
          pltpu.SemaphoreType.DMA,
      ],
  )
  return kern(cfg, zcat, src_t, dst_t, zer)


def _l1_body(x_ref, d_ref, w_ref, o_ref):
  ns = lax.rsqrt(jnp.maximum(d_ref[...], 1.0))
  z = jnp.dot(x_ref[...] * ns, w_ref[...], preferred_element_type=jnp.float32)
  for c in range(NCK):
    o_ref[c] = z[:, c * CW:(c + 1) * CW]


def _layer1(x_pad, dout_col, w1):
  return pl.pallas_call(
      _l1_body,
      grid=(NP // RPT,),
      in_specs=[
          pl.BlockSpec((RPT, DIN), lambda i: (i, 0)),
          pl.BlockSpec((RPT, 1), lambda i: (i, 0)),
          pl.BlockSpec((DIN, HH), lambda i: (0, 0)),
      ],
      out_specs=pl.BlockSpec((NCK, RPT, CW), lambda i: (0, i, 0)),
      out_shape=jax.ShapeDtypeStruct((NCK, NP, CW), jnp.float32),
  )(x_pad, dout_col, w1)


def _mid_body(m_ref, din_ref, dout_ref, b_ref, w_ref, o_ref):
  nd = lax.rsqrt(jnp.maximum(din_ref[...], 1.0))
  ns = lax.rsqrt(jnp.maximum(dout_ref[...], 1.0))
  m = jnp.concatenate([m_ref[c] for c in range(NCK)], axis=1)
  h = jnp.maximum(m * nd + b_ref[...], 0.0) * ns
  z = jnp.dot(h, w_ref[...], preferred_element_type=jnp.float32)
  for c in range(NCK):
    o_ref[c] = z[:, c * CW:(c + 1) * CW]


def _layer_mid(m_cat, din_col, dout_col, b_row, w):
  return pl.pallas_call(
      _mid_body,
      grid=(NP // RPT,),
      in_specs=[
          pl.BlockSpec((NCK, RPT, CW), lambda i: (0, i, 0)),
          pl.BlockSpec((RPT, 1), lambda i: (i, 0)),
          pl.BlockSpec((RPT, 1), lambda i: (i, 0)),
          pl.BlockSpec((1, HH), lambda i: (0, 0)),
          pl.BlockSpec((HH, HH), lambda i: (0, 0)),
      ],
      out_specs=pl.BlockSpec((NCK, RPT, CW), lambda i: (0, i, 0)),
      out_shape=jax.ShapeDtypeStruct((NCK, NP, CW), jnp.float32),
  )(m_cat, din_col, dout_col, b_row, w)


def _l4_body(m_ref, din_ref, dout_ref, b_ref, w_ref, o_ref):
  nd = lax.rsqrt(jnp.maximum(din_ref[...], 1.0))
  ns = lax.rsqrt(jnp.maximum(dout_ref[...], 1.0))
  m = jnp.concatenate([m_ref[c] for c in range(NCK)], axis=1)
  h = jnp.maximum(m * nd + b_ref[...], 0.0) * ns
  o_ref[...] = jnp.dot(h, w_ref[...], preferred_element_type=jnp.float32)


def _layer4(m_cat, din_col, dout_col, b_row, w4p):
  return pl.pallas_call(
      _l4_body,
      grid=(NP // RPT,),
      in_specs=[
          pl.BlockSpec((NCK, RPT, CW), lambda i: (0, i, 0)),
          pl.BlockSpec((RPT, 1), lambda i: (i, 0)),
          pl.BlockSpec((RPT, 1), lambda i: (i, 0)),
          pl.BlockSpec((1, HH), lambda i: (0, 0)),
          pl.BlockSpec((HH, CW), lambda i: (0, 0)),
      ],
      out_specs=pl.BlockSpec((RPT, CW), lambda i: (i, 0)),
      out_shape=jax.ShapeDtypeStruct((NP, CW), jnp.float32),
  )(m_cat, din_col, dout_col, b_row, w4p)


def _fin_body(p_ref, din_ref, b_ref, o_ref):
  nd = lax.rsqrt(jnp.maximum(din_ref[...], 1.0))
  p = p_ref[0] + p_ref[1] + p_ref[2] + p_ref[3]
  o_ref[...] = p[:, :CC] * nd + b_ref[...]


def _final(parts, din_col, b4_row):
  return pl.pallas_call(
      _fin_body,
      grid=(NP // RPT,),
      in_specs=[
          pl.BlockSpec((NCK, RPT, CW), lambda i: (0, i, 0)),
          pl.BlockSpec((RPT, 1), lambda i: (i, 0)),
          pl.BlockSpec((1, CC), lambda i: (0, 0)),
      ],
      out_specs=pl.BlockSpec((RPT, CC), lambda i: (i, 0)),
      out_shape=jax.ShapeDtypeStruct((NP, CC), jnp.float32),
  )(parts, din_col, b4_row)


@jax.jit
def kernel(x, edge_index, W1, b1, W2, b2, W3, b3, W4, b4):
  src = edge_index[0]
  dst = edge_index[1]
  pad = EP - EE

  src_deg = jnp.concatenate(
      [src, jnp.full((pad,), NN, jnp.int32)]).reshape(NT, NB, BE)
  dst_pad = jnp.concatenate(
      [dst, jnp.full((pad,), NN, jnp.int32)]).reshape(NT, NB, BE)
  src_pad = jnp.concatenate([src, jnp.zeros((pad,), jnp.int32)])
  offs = (jnp.arange(NCK, dtype=jnp.int32) * NP)[:, None]
  src4 = (src_pad[None, :] + offs).reshape(NCK, NT, NB, BE)
  src_fin = jnp.broadcast_to(src_pad.reshape(NT, NB, BE), (NCK, NT, NB, BE))
  dst4 = jnp.broadcast_to(dst_pad, (NCK, NT, NB, BE))

  zer = jnp.zeros((RPT, CW), jnp.float32)
  x_pad = jnp.concatenate([x, jnp.zeros((NP - NN, DIN), jnp.float32)])
  w4p = jnp.pad(W4, ((0, 0), (0, CW - CC)))

  def _cfg(rows):
    return jnp.broadcast_to(
        jnp.array(rows, jnp.int32)[:, :, None], (NCK, 2, 16))

  cfg_full = _cfg([[0, NSB]] * NCK)
  cfg_deg = _cfg([[0, NSB], [0, 0], [0, NSB], [0, 0]])
  cfg_fin = _cfg([[0, NSB // 2], [0, 0], [NSB // 2, NSB], [0, 0]])

  ones_cat = jnp.ones((NCK * NP, CW), jnp.float32)
  dst_deg4 = jnp.stack([src_deg, src_deg, dst_pad, dst_pad])
  deg = _edge_agg(cfg_deg, ones_cat, src4, dst_deg4, zer)
  dout_col = deg[0, :, 0:1]
  din_col = deg[2, :, 0:1]

  z1 = _layer1(x_pad, dout_col, W1).reshape(NCK * NP, CW)
  m1 = _edge_agg(cfg_full, z1, src4, dst4, zer)
  z2 = _layer_mid(m1, din_col, dout_col, b1.reshape(1, HH),
                  W2).reshape(NCK * NP, CW)
  m2 = _edge_agg(cfg_full, z2, src4, dst4, zer)
  z3 = _layer_mid(m2, din_col, dout_col, b2.reshape(1, HH),
                  W3).reshape(NCK * NP, CW)
  m3 = _edge_agg(cfg_full, z3, src4, dst4, zer)
  z4 = _layer4(m3, din_col, dout_col, b3.reshape(1, HH), w4p)
  z4cat = jnp.concatenate(
      [z4, jnp.zeros(((NCK - 1) * NP, CW), jnp.float32)])
  parts = _edge_agg(cfg_fin, z4cat, src_fin, dst4, zer)
  out = _final(parts, din_col, b4.reshape(1, CC))
  return out[:NN]


# speedup vs baseline: 1.0609x; 1.0005x over previous
import dataclasses

import jax
import jax.numpy as jnp
from jax import lax
from jax.experimental import pallas as pl
from jax.experimental.pallas import tpu as pltpu
from jax.experimental.pallas import tpu_sc as plsc

NN = 10000
NP = 10240
RPT = 640
EE = 320000
NT = 16
NB = 160
BE = 128
EP = NT * NB * BE
SB = 16
NSB = NB // SB
DIN = 128
HH = 512
CC = 64
CW = 128
NCK = HH // CW


def _agg_body(cfg_h, zcat_h, src_h, dst_h, zer_h, out_h, cfg_s, si0, si1,
              didx_v, b0, b1, acc_s, s0, s1, is0, is1):
  bufs = (b0, b1)
  sems = (s0, s1)
  sidx = (si0, si1)
  isems = (is0, is1)
  core = lax.axis_index("c")
  tid = lax.axis_index("s")
  row0 = tid * RPT

  pltpu.sync_copy(cfg_h, cfg_s)
  for ci in range(2):
    slot = core * 2 + ci
    lo = jnp.max(cfg_s[slot, 0])
    hi = jnp.max(cfg_s[slot, 1])
    pltpu.sync_copy(zer_h, acc_s.at[pl.ds(row0, RPT)])
    plsc.subcore_barrier()

    @pl.when(hi > lo)
    def _():
      pltpu.sync_copy(src_h.at[slot, tid, pl.ds(lo * SB, SB)], sidx[0])

    @pl.loop(0, (hi - lo) // 2)
    def _(p):
      for par in range(2):
        sb = lo + 2 * p + par

        @pl.when(sb > lo)
        def _():
          pltpu.make_async_copy(src_h.at[slot, tid, pl.ds(0, SB)],
                                sidx[par], isems[par]).wait()

        sv = sidx[par]
        for b in range(2):
          pltpu.async_copy(zcat_h.at[sv.at[b]], bufs[b], sems[b])

        @pl.when(sb + 1 < hi)
        def _():
          pltpu.async_copy(src_h.at[slot, tid, pl.ds((sb + 1) * SB, SB)],
                           sidx[1 - par], isems[1 - par])

        pltpu.sync_copy(dst_h.at[slot, tid, pl.ds(sb * SB, SB)], didx_v)
        for k in range(SB):
          b = k % 2
          pltpu.make_async_copy(zcat_h.at[sv.at[0]], bufs[b], sems[b]).wait()
          pltpu.sync_copy(bufs[b], acc_s.at[didx_v.at[k]], add=True)
          if k + 2 < SB:
            pltpu.async_copy(zcat_h.at[sv.at[k + 2]], bufs[b], sems[b])

    @pl.when(jnp.logical_and(hi > lo, (hi - lo) % 2 == 1))
    def _():
      sb = hi - 1

      @pl.when(sb > lo)
      def _():
        pltpu.make_async_copy(src_h.at[slot, tid, pl.ds(0, SB)], sidx[0],
                              isems[0]).wait()

      sv = sidx[0]
      for b in range(2):
        pltpu.async_copy(zcat_h.at[sv.at[b]], bufs[b], sems[b])
      pltpu.sync_copy(dst_h.at[slot, tid, pl.ds(sb * SB, SB)], didx_v)
      for k in range(SB):
        b = k % 2
        pltpu.make_async_copy(zcat_h.at[sv.at[0]], bufs[b], sems[b]).wait()
        pltpu.sync_copy(bufs[b], acc_s.at[didx_v.at[k]], add=True)
        if k + 2 < SB:
          pltpu.async_copy(zcat_h.at[sv.at[k + 2]], bufs[b], sems[b])

    plsc.subcore_barrier()
    pltpu.sync_copy(acc_s.at[pl.ds(row0, RPT)],
                    out_h.at[slot, pl.ds(row0, RPT)])
    plsc.subcore_barrier()


def _sc_params():
  cp = pltpu.CompilerParams()
  if "needs_layout_passes" in pltpu.CompilerParams.__dataclass_fields__:
    cp = dataclasses.replace(cp, needs_layout_passes=False)
  return cp


def _edge_agg(cfg, zcat, src_t, dst_t, zer):
  kern = pl.kernel(
      _agg_body,
      compiler_params=_sc_params(),
      out_type=jax.ShapeDtypeStruct((NCK, NP, CW), jnp.float32),
      mesh=plsc.VectorSubcoreMesh(core_axis_name="c", subcore_axis_name="s"),
      scratch_types=[
          pltpu.VMEM((NCK, 2, 16), jnp.int32),
          pltpu.VMEM((SB, BE), jnp.int32),
          pltpu.VMEM((SB, BE), jnp.int32),
          pltpu.VMEM((SB, BE), jnp.int32),
          pltpu.VMEM((BE, CW), jnp.float32),
          pltpu.VMEM((BE, CW), jnp.float32),
          pltpu.VMEM_SHARED((NP, CW), jnp.float32),
          pltpu.SemaphoreType.DMA,
          pltpu.SemaphoreType.DMA,
          pltpu.SemaphoreType.DMA,
          pltpu.SemaphoreType.DMA,
      ],
  )
  return kern(cfg, zcat, src_t, dst_t, zer)


def _l1_body(x_ref, d_ref, w_ref, o_ref):
  ns = lax.rsqrt(jnp.maximum(d_ref[...], 1.0))
  z = jnp.dot(x_ref[...] * ns, w_ref[...], preferred_element_type=jnp.float32)
  for c in range(NCK):
    o_ref[c] = z[:, c * CW:(c + 1) * CW]


def _layer1(x_pad, dout_col, w1):
  return pl.pallas_call(
      _l1_body,
      grid=(NP // RPT,),
      in_specs=[
          pl.BlockSpec((RPT, DIN), lambda i: (i, 0)),
          pl.BlockSpec((RPT, 1), lambda i: (i, 0)),
          pl.BlockSpec((DIN, HH), lambda i: (0, 0)),
      ],
      out_specs=pl.BlockSpec((NCK, RPT, CW), lambda i: (0, i, 0)),
      out_shape=jax.ShapeDtypeStruct((NCK, NP, CW), jnp.float32),
  )(x_pad, dout_col, w1)


def _mid_body(m_ref, din_ref, dout_ref, b_ref, w_ref, o_ref):
  nd = lax.rsqrt(jnp.maximum(din_ref[...], 1.0))
  ns = lax.rsqrt(jnp.maximum(dout_ref[...], 1.0))
  m = jnp.concatenate([m_ref[c] for c in range(NCK)], axis=1)
  h = jnp.maximum(m * nd + b_ref[...], 0.0) * ns
  z = jnp.dot(h, w_ref[...], preferred_element_type=jnp.float32)
  for c in range(NCK):
    o_ref[c] = z[:, c * CW:(c + 1) * CW]


def _layer_mid(m_cat, din_col, dout_col, b_row, w):
  return pl.pallas_call(
      _mid_body,
      grid=(NP // RPT,),
      in_specs=[
          pl.BlockSpec((NCK, RPT, CW), lambda i: (0, i, 0)),
          pl.BlockSpec((RPT, 1), lambda i: (i, 0)),
          pl.BlockSpec((RPT, 1), lambda i: (i, 0)),
          pl.BlockSpec((1, HH), lambda i: (0, 0)),
          pl.BlockSpec((HH, HH), lambda i: (0, 0)),
      ],
      out_specs=pl.BlockSpec((NCK, RPT, CW), lambda i: (0, i, 0)),
      out_shape=jax.ShapeDtypeStruct((NCK, NP, CW), jnp.float32),
  )(m_cat, din_col, dout_col, b_row, w)


def _l4_body(m_ref, din_ref, dout_ref, b_ref, w_ref, o_ref):
  nd = lax.rsqrt(jnp.maximum(din_ref[...], 1.0))
  ns = lax.rsqrt(jnp.maximum(dout_ref[...], 1.0))
  m = jnp.concatenate([m_ref[c] for c in range(NCK)], axis=1)
  h = jnp.maximum(m * nd + b_ref[...], 0.0) * ns
  o_ref[...] = jnp.dot(h, w_ref[...], preferred_element_type=jnp.float32)


def _layer4(m_cat, din_col, dout_col, b_row, w4p):
  return pl.pallas_call(
      _l4_body,
      grid=(NP // RPT,),
      in_specs=[
          pl.BlockSpec((NCK, RPT, CW), lambda i: (0, i, 0)),
          pl.BlockSpec((RPT, 1), lambda i: (i, 0)),
          pl.BlockSpec((RPT, 1), lambda i: (i, 0)),
          pl.BlockSpec((1, HH), lambda i: (0, 0)),
          pl.BlockSpec((HH, CW), lambda i: (0, 0)),
      ],
      out_specs=pl.BlockSpec((RPT, CW), lambda i: (i, 0)),
      out_shape=jax.ShapeDtypeStruct((NP, CW), jnp.float32),
  )(m_cat, din_col, dout_col, b_row, w4p)


def _fin_body(p_ref, din_ref, b_ref, o_ref):
  nd = lax.rsqrt(jnp.maximum(din_ref[...], 1.0))
  p = p_ref[0] + p_ref[1] + p_ref[2] + p_ref[3]
  o_ref[...] = p[:, :CC] * nd + b_ref[...]


def _final(parts, din_col, b4_row):
  return pl.pallas_call(
      _fin_body,
      grid=(NP // RPT,),
      in_specs=[
          pl.BlockSpec((NCK, RPT, CW), lambda i: (0, i, 0)),
          pl.BlockSpec((RPT, 1), lambda i: (i, 0)),
          pl.BlockSpec((1, CC), lambda i: (0, 0)),
      ],
      out_specs=pl.BlockSpec((RPT, CC), lambda i: (i, 0)),
      out_shape=jax.ShapeDtypeStruct((NP, CC), jnp.float32),
  )(parts, din_col, b4_row)


@jax.jit
def kernel(x, edge_index, W1, b1, W2, b2, W3, b3, W4, b4):
  src = edge_index[0]
  dst = edge_index[1]
  pad = EP - EE

  bucket = NN + (jnp.arange(pad, dtype=jnp.int32) % (NP - NN))
  src_deg = jnp.concatenate([src, bucket]).reshape(NT, NB, BE)
  dst_pad = jnp.concatenate([dst, bucket]).reshape(NT, NB, BE)
  src_pad = jnp.concatenate([src, jnp.zeros((pad,), jnp.int32)])
  offs = (jnp.arange(NCK, dtype=jnp.int32) * NP)[:, None]
  src4 = (src_pad[None, :] + offs).reshape(NCK, NT, NB, BE)
  src_fin = jnp.broadcast_to(src_pad.reshape(NT, NB, BE), (NCK, NT, NB, BE))
  dst4 = jnp.broadcast_to(dst_pad, (NCK, NT, NB, BE))

  zer = jnp.zeros((RPT, CW), jnp.float32)
  x_pad = jnp.concatenate([x, jnp.zeros((NP - NN, DIN), jnp.float32)])
  w4p = jnp.pad(W4, ((0, 0), (0, CW - CC)))

  def _cfg(rows):
    return jnp.broadcast_to(
        jnp.array(rows, jnp.int32)[:, :, None], (NCK, 2, 16))

  cfg_full = _cfg([[0, NSB]] * NCK)
  cfg_deg = _cfg([[0, NSB], [0, 0], [0, NSB], [0, 0]])
  cfg_fin = _cfg([[0, NSB // 2], [0, 0], [NSB // 2, NSB], [0, 0]])

  ones_cat = jnp.ones((NCK * NP, CW), jnp.float32)
  dst_deg4 = jnp.stack([src_deg, src_deg, dst_pad, dst_pad])
  deg = _edge_agg(cfg_deg, ones_cat, src4, dst_deg4, zer)
  dout_col = deg[0, :, 0:1]
  din_col = deg[2, :, 0:1]

  z1 = _layer1(x_pad, dout_col, W1).reshape(NCK * NP, CW)
  m1 = _edge_agg(cfg_full, z1, src4, dst4, zer)
  z2 = _layer_mid(m1, din_col, dout_col, b1.reshape(1, HH),
                  W2).reshape(NCK * NP, CW)
  m2 = _edge_agg(cfg_full, z2, src4, dst4, zer)
  z3 = _layer_mid(m2, din_col, dout_col, b2.reshape(1, HH),
                  W3).reshape(NCK * NP, CW)
  m3 = _edge_agg(cfg_full, z3, src4, dst4, zer)
  z4 = _layer4(m3, din_col, dout_col, b3.reshape(1, HH), w4p)
  z4cat = jnp.concatenate(
      [z4, jnp.zeros(((NCK - 1) * NP, CW), jnp.float32)])
  parts = _edge_agg(cfg_fin, z4cat, src_fin, dst4, zer)
  out = _final(parts, din_col, b4.reshape(1, CC))
  return out[:NN]
